# Initial kernel scaffold; baseline (speedup 1.0000x reference)
import jax
import jax.numpy as jnp
from jax.experimental import pallas as pl


def kernel(x, Ws, bs, Wh, bh, Wout, bout, nn2_params, nn3_params):
    raise NotImplementedError("write your pallas kernel here")


# speedup vs baseline: 3.0672x; 3.0672x over previous
import functools

import jax
import jax.numpy as jnp
from jax import lax
from jax.experimental import pallas as pl
from jax.experimental.pallas import tpu as pltpu
from jax.experimental.pallas import tpu_sc as plsc

_N = 10000
_NP = 10240
_K = 16
_HID = 32
_R = 256
_CW = 1024
_NCK = _NP // _CW
_RT = 512
_CH = 8
_HW = 128


def _dot(a, b):
    return jnp.dot(a, b, preferred_element_type=jnp.float32)


def _proj_body(x_ref, ws_ref, bs_ref, wh_ref, bh_ref, s_ref, h_ref, sq_ref):
    x = x_ref[...]
    s = _dot(x, ws_ref[...]) + bs_ref[...]
    s_ref[...] = s
    h_ref[...] = _dot(x, wh_ref[...]) + bh_ref[...]
    sq_ref[...] = jnp.sum(s * s, axis=1, keepdims=True)


def _proj(xp, Ws, bs, Wh, bh):
    grid = (_NP // _RT,)
    return pl.pallas_call(
        _proj_body,
        grid=grid,
        in_specs=[
            pl.BlockSpec((_RT, 15), lambda i: (i, 0)),
            pl.BlockSpec(Ws.shape, lambda i: (0, 0)),
            pl.BlockSpec(bs.shape, lambda i: (0, 0)),
            pl.BlockSpec(Wh.shape, lambda i: (0, 0)),
            pl.BlockSpec(bh.shape, lambda i: (0, 0)),
        ],
        out_specs=[
            pl.BlockSpec((_RT, 4), lambda i: (i, 0)),
            pl.BlockSpec((_RT, _HW), lambda i: (i, 0)),
            pl.BlockSpec((_RT, 1), lambda i: (i, 0)),
        ],
        out_shape=(
            jax.ShapeDtypeStruct((_NP, 4), jnp.float32),
            jax.ShapeDtypeStruct((_NP, _HW), jnp.float32),
            jax.ShapeDtypeStruct((_NP, 1), jnp.float32),
        ),
    )(xp, Ws, bs, Wh, bh)


def _knn_body(sq_ref, sq3_ref, s_ref, st3_ref, idx_ref, w_ref, d2_ref):
    s_tile = s_ref[...]
    sq_tile = sq_ref[...]
    big = jnp.float32(jnp.inf)

    def fill(c, carry):
        stc = st3_ref[c]
        d2c = sq_tile + sq3_ref[c] - 2.0 * _dot(s_tile, stc)
        colc = c * _CW + lax.broadcasted_iota(jnp.int32, (_R, _CW), 1)
        d2_ref[c] = jnp.where(colc < _N, d2c, big)
        return carry

    lax.fori_loop(0, _NCK, fill, 0)

    idxs, vals = [], []
    sel_prev = jnp.full((_R, 1), -1, jnp.int32)
    for _ in range(_K):
        def p_a(c, m, sel_prev=sel_prev):
            d2c = d2_ref[c]
            colc = c * _CW + lax.broadcasted_iota(jnp.int32, (_R, _CW), 1)
            d2c = jnp.where(colc == sel_prev, big, d2c)
            d2_ref[c] = d2c
            return jnp.minimum(m, jnp.min(d2c, axis=1, keepdims=True))

        m = lax.fori_loop(0, _NCK, p_a, jnp.full((_R, 1), big))

        def p_b(c, sel, m=m):
            d2c = d2_ref[c]
            colc = c * _CW + lax.broadcasted_iota(jnp.int32, (_R, _CW), 1)
            selc = jnp.min(jnp.where(d2c <= m, colc, _NP), axis=1,
                           keepdims=True)
            return jnp.minimum(sel, selc)

        sel = lax.fori_loop(0, _NCK, p_b,
                            jnp.full((_R, 1), _NP, jnp.int32))
        vals.append(m)
        idxs.append(sel)
        sel_prev = sel
    idx_ref[...] = jnp.concatenate(idxs, axis=1)
    w_ref[...] = jnp.exp(-10.0 * jnp.maximum(jnp.concatenate(vals, axis=1),
                                             0.0))


def _knn(sq, sq3, s, sT3):
    grid = (_NP // _R,)
    return pl.pallas_call(
        _knn_body,
        grid=grid,
        in_specs=[
            pl.BlockSpec((_R, 1), lambda i: (i, 0)),
            pl.BlockSpec((_NCK, 1, _CW), lambda i: (0, 0, 0)),
            pl.BlockSpec((_R, 4), lambda i: (i, 0)),
            pl.BlockSpec((_NCK, 4, _CW), lambda i: (0, 0, 0)),
        ],
        out_specs=[
            pl.BlockSpec((_R, _K), lambda i: (i, 0)),
            pl.BlockSpec((_R, _K), lambda i: (i, 0)),
        ],
        out_shape=(
            jax.ShapeDtypeStruct((_NP, _K), jnp.int32),
            jax.ShapeDtypeStruct((_NP, _K), jnp.float32),
        ),
        scratch_shapes=[pltpu.VMEM((_NCK, _R, _CW), jnp.float32)],
    )(sq, sq3, s, sT3)


def _agg_sc(h, idx_flat, w):
    mesh = plsc.VectorSubcoreMesh(core_axis_name="c", subcore_axis_name="s")
    npt = _NP // 32
    nchunks = npt // _CH

    @functools.partial(
        pl.kernel,
        mesh=mesh,
        out_type=jax.ShapeDtypeStruct((_NP, 2 * _HID), jnp.float32),
        scratch_types=[
            pltpu.VMEM((_CH * _K,), jnp.int32),
            pltpu.VMEM((_CH, _K), jnp.float32),
            pltpu.VMEM((_CH * _K, _HW), jnp.float32),
            pltpu.VMEM((_CH, 2 * _HID), jnp.float32),
            pltpu.SemaphoreType.DMA,
        ],
    )
    def body(h_hbm, idx_hbm, w_hbm, out_hbm, idx_v, w_v, rows_v, out_v, sem):
        wid = lax.axis_index("s") * 2 + lax.axis_index("c")
        base = wid * npt

        def chunk(ci, carry):
            r0 = base + ci * _CH
            pltpu.sync_copy(idx_hbm.at[pl.ds(r0 * _K, _CH * _K)], idx_v)
            pltpu.async_copy(h_hbm.at[idx_v], rows_v, sem).wait()
            pltpu.sync_copy(w_hbm.at[pl.ds(r0, _CH)], w_v)
            for n in range(_CH):
                acc0 = jnp.zeros((16,), jnp.float32)
                acc1 = jnp.zeros((16,), jnp.float32)
                mx0 = jnp.full((16,), -jnp.inf, jnp.float32)
                mx1 = jnp.full((16,), -jnp.inf, jnp.float32)
                wrow = w_v[n, 0:16]
                for j in range(_K):
                    r = n * _K + j
                    wv = wrow[j]
                    a0 = rows_v[r, 0:16] * wv
                    a1 = rows_v[r, 16:32] * wv
                    acc0 = acc0 + a0
                    acc1 = acc1 + a1
                    mx0 = jnp.maximum(mx0, a0)
                    mx1 = jnp.maximum(mx1, a1)
                out_v[n, 0:16] = acc0 * (1.0 / _K)
                out_v[n, 16:32] = acc1 * (1.0 / _K)
                out_v[n, 32:48] = mx0
                out_v[n, 48:64] = mx1
            pltpu.sync_copy(out_v, out_hbm.at[pl.ds(r0, _CH)])
            return carry

        lax.fori_loop(0, nchunks, chunk, 0)

    return body(h, idx_flat, w)


def _mlp_body(x_ref, agg_ref, *refs):
    wr = [r[...] for r in refs[:-2]]
    ids_ref, p4_ref = refs[-2], refs[-1]
    (Wox, Wom, WoM, bo,
     W2x, W2h, b20, W21, b21, W22, b22, W23, b23, W24, b24,
     W3x, W3h, W3i, b30, W31, b31, W32, b32, W33, b33, W34, b34) = wr
    x = x_ref[...]
    mean = agg_ref[:, 0:_HID]
    mx = agg_ref[:, _HID:2 * _HID]

    def leaky(v):
        return jnp.where(v >= 0, v, 0.01 * v)

    x1 = leaky(_dot(x, Wox) + _dot(mean, Wom) + _dot(mx, WoM) + bo)
    a = leaky(_dot(x, W2x) + _dot(x1, W2h) + b20)
    a = leaky(_dot(a, W21) + b21)
    a = leaky(_dot(a, W22) + b22)
    a = leaky(_dot(a, W23) + b23)
    ids = _dot(a, W24) + b24
    c = leaky(_dot(x, W3x) + _dot(x1, W3h) + _dot(ids, W3i) + b30)
    c = leaky(_dot(c, W31) + b31)
    c = leaky(_dot(c, W32) + b32)
    c = leaky(_dot(c, W33) + b33)
    o3 = _dot(c, W34) + b34
    ids_ref[...] = ids
    p4_ref[...] = x[:, 11:15] + o3


def _mlp(xp, agg, weights):
    grid = (_NP // _RT,)
    full = [pl.BlockSpec(w.shape, lambda i: (0,) * w.ndim) for w in weights]
    return pl.pallas_call(
        _mlp_body,
        grid=grid,
        in_specs=[
            pl.BlockSpec((_RT, 15), lambda i: (i, 0)),
            pl.BlockSpec((_RT, 2 * _HID), lambda i: (i, 0)),
        ] + full,
        out_specs=[
            pl.BlockSpec((_RT, 6), lambda i: (i, 0)),
            pl.BlockSpec((_RT, 4), lambda i: (i, 0)),
        ],
        out_shape=(
            jax.ShapeDtypeStruct((_NP, 6), jnp.float32),
            jax.ShapeDtypeStruct((_NP, 4), jnp.float32),
        ),
    )(xp, agg, *weights)


def kernel(x, Ws, bs, Wh, bh, Wout, bout, nn2_params, nn3_params):
    xp = jnp.pad(x, ((0, _NP - _N), (0, 0)))
    Whp = jnp.pad(Wh, ((0, 0), (0, _HW - _HID)))
    bhp = jnp.pad(bh, (0, _HW - _HID))
    s, h, sq = _proj(xp, Ws, bs.reshape(1, -1), Whp, bhp.reshape(1, -1))
    sT3 = s.T.reshape(4, _NCK, _CW).swapaxes(0, 1)
    sq3 = sq.reshape(1, _NCK, _CW).swapaxes(0, 1)
    idx, w = _knn(sq, sq3, s, sT3)
    agg = _agg_sc(h, idx.reshape(-1), w)

    W20, b20 = nn2_params[0], nn2_params[1]
    W30, b30 = nn3_params[0], nn3_params[1]
    weights = [
        Wout[:15], Wout[15:15 + _HID], Wout[15 + _HID:15 + 2 * _HID],
        bout.reshape(1, -1),
        W20[:15], W20[15:], b20.reshape(1, -1),
        nn2_params[2], nn2_params[3].reshape(1, -1),
        nn2_params[4], nn2_params[5].reshape(1, -1),
        nn2_params[6], nn2_params[7].reshape(1, -1),
        nn2_params[8], nn2_params[9].reshape(1, -1),
        W30[:15], W30[15:271], W30[271:277], b30.reshape(1, -1),
        nn3_params[2], nn3_params[3].reshape(1, -1),
        nn3_params[4], nn3_params[5].reshape(1, -1),
        nn3_params[6], nn3_params[7].reshape(1, -1),
        nn3_params[8], nn3_params[9].reshape(1, -1),
    ]
    ids, p4 = _mlp(xp, agg, weights)
    return ids[:_N], p4[:_N]


# speedup vs baseline: 5.0258x; 1.6386x over previous
import functools

import jax
import jax.numpy as jnp
from jax import lax
from jax.experimental import pallas as pl
from jax.experimental.pallas import tpu as pltpu
from jax.experimental.pallas import tpu_sc as plsc

_N = 10000
_NP = 10240
_K = 16
_HID = 32
_R = 256
_CW = 1024
_NCK = _NP // _CW
_RT = 512
_CH = 8
_HW = 128


def _dot(a, b):
    return jnp.dot(a, b, preferred_element_type=jnp.float32)


def _proj_body(x_ref, ws_ref, bs_ref, wh_ref, bh_ref, s_ref, h_ref, sq_ref):
    x = x_ref[...]
    s = _dot(x, ws_ref[...]) + bs_ref[...]
    s_ref[...] = s
    h_ref[...] = _dot(x, wh_ref[...]) + bh_ref[...]
    sq_ref[...] = jnp.sum(s * s, axis=1, keepdims=True)


def _proj(xp, Ws, bs, Wh, bh):
    grid = (_NP // _RT,)
    return pl.pallas_call(
        _proj_body,
        grid=grid,
        in_specs=[
            pl.BlockSpec((_RT, 15), lambda i: (i, 0)),
            pl.BlockSpec(Ws.shape, lambda i: (0, 0)),
            pl.BlockSpec(bs.shape, lambda i: (0, 0)),
            pl.BlockSpec(Wh.shape, lambda i: (0, 0)),
            pl.BlockSpec(bh.shape, lambda i: (0, 0)),
        ],
        out_specs=[
            pl.BlockSpec((_RT, 4), lambda i: (i, 0)),
            pl.BlockSpec((_RT, _HW), lambda i: (i, 0)),
            pl.BlockSpec((_RT, 1), lambda i: (i, 0)),
        ],
        out_shape=(
            jax.ShapeDtypeStruct((_NP, 4), jnp.float32),
            jax.ShapeDtypeStruct((_NP, _HW), jnp.float32),
            jax.ShapeDtypeStruct((_NP, 1), jnp.float32),
        ),
    )(xp, Ws, bs, Wh, bh)


def _knn_body(sq_ref, sq3_ref, s_ref, st3_ref, idx_ref, w_ref, d2_ref):
    s_tile = s_ref[...]
    sq_tile = sq_ref[...]
    big = jnp.float32(jnp.inf)

    mfull = jnp.full((_R, _CW), big)
    afull = jnp.full((_R, _CW), _NP, jnp.int32)

    def casc(c, carry):
        m1, m2, m3, m4, m5, a1, a2, a3, a4, a5 = carry
        stc = st3_ref[c]
        colc = c * _CW + lax.broadcasted_iota(jnp.int32, (_R, _CW), 1)
        x = sq_tile + sq3_ref[c] - 2.0 * _dot(s_tile, stc)
        x = jnp.where(colc < _N, x, big)
        d2_ref[c] = x
        c1 = x < m1
        c2 = x < m2
        c3 = x < m3
        c4 = x < m4
        c5 = x < m5
        m5n = jnp.where(c5, jnp.where(c4, m4, x), m5)
        a5n = jnp.where(c5, jnp.where(c4, a4, colc), a5)
        m4n = jnp.where(c4, jnp.where(c3, m3, x), m4)
        a4n = jnp.where(c4, jnp.where(c3, a3, colc), a4)
        m3n = jnp.where(c3, jnp.where(c2, m2, x), m3)
        a3n = jnp.where(c3, jnp.where(c2, a2, colc), a3)
        m2n = jnp.where(c2, jnp.where(c1, m1, x), m2)
        a2n = jnp.where(c2, jnp.where(c1, a1, colc), a2)
        m1n = jnp.where(c1, x, m1)
        a1n = jnp.where(c1, colc, a1)
        return (m1n, m2n, m3n, m4n, m5n, a1n, a2n, a3n, a4n, a5n)

    m1, m2, m3, m4, m5, a1, a2, a3, a4, a5 = lax.fori_loop(
        0, _NCK, casc,
        (mfull, mfull, mfull, mfull, mfull, afull, afull, afull, afull,
         afull))

    cnt = jnp.zeros((_R, _CW), jnp.int32)
    idxs, vals = [], []
    for _ in range(_K):
        m = jnp.min(m1, axis=1, keepdims=True)
        idxt = jnp.min(jnp.where(m1 <= m, a1, _NP), axis=1, keepdims=True)
        gmask = a1 == idxt
        vals.append(m)
        idxs.append(idxt)
        m1 = jnp.where(gmask, m2, m1)
        a1 = jnp.where(gmask, a2, a1)
        m2 = jnp.where(gmask, m3, m2)
        a2 = jnp.where(gmask, a3, a2)
        m3 = jnp.where(gmask, m4, m3)
        a3 = jnp.where(gmask, a4, a3)
        m4 = jnp.where(gmask, m5, m4)
        a4 = jnp.where(gmask, a5, a4)
        m5 = jnp.where(gmask, big, m5)
        a5 = jnp.where(gmask, _NP, a5)
        cnt = cnt + gmask.astype(jnp.int32)
    idx_ref[...] = jnp.concatenate(idxs, axis=1)
    w_ref[...] = jnp.exp(-10.0 * jnp.maximum(jnp.concatenate(vals, axis=1),
                                             0.0))

    @pl.when(jnp.any(cnt >= 5))
    def _slow():
        sidxs, svals = [], []
        sel_prev = jnp.full((_R, 1), -1, jnp.int32)
        for t in range(_K):
            def p_a(c, mm, sel_prev=sel_prev):
                d2c = d2_ref[c]
                colc = c * _CW + lax.broadcasted_iota(jnp.int32,
                                                      (_R, _CW), 1)
                d2c = jnp.where(colc == sel_prev, big, d2c)
                d2_ref[c] = d2c
                return jnp.minimum(mm, jnp.min(d2c, axis=1, keepdims=True))

            mm = lax.fori_loop(0, _NCK, p_a, jnp.full((_R, 1), big))

            def p_b(c, sel, mm=mm):
                d2c = d2_ref[c]
                colc = c * _CW + lax.broadcasted_iota(jnp.int32,
                                                      (_R, _CW), 1)
                selc = jnp.min(jnp.where(d2c <= mm, colc, _NP), axis=1,
                               keepdims=True)
                return jnp.minimum(sel, selc)

            sel = lax.fori_loop(0, _NCK, p_b,
                                jnp.full((_R, 1), _NP, jnp.int32))
            svals.append(mm)
            sidxs.append(sel)
            sel_prev = sel
        idx_ref[...] = jnp.concatenate(sidxs, axis=1)
        w_ref[...] = jnp.exp(
            -10.0 * jnp.maximum(jnp.concatenate(svals, axis=1), 0.0))
    idx_ref[...] = jnp.concatenate(idxs, axis=1)
    w_ref[...] = jnp.exp(-10.0 * jnp.maximum(jnp.concatenate(vals, axis=1),
                                             0.0))


def _knn(sq, sq3, s, sT3):
    grid = (_NP // _R,)
    return pl.pallas_call(
        _knn_body,
        grid=grid,
        in_specs=[
            pl.BlockSpec((_R, 1), lambda i: (i, 0)),
            pl.BlockSpec((_NCK, 1, _CW), lambda i: (0, 0, 0)),
            pl.BlockSpec((_R, 4), lambda i: (i, 0)),
            pl.BlockSpec((_NCK, 4, _CW), lambda i: (0, 0, 0)),
        ],
        out_specs=[
            pl.BlockSpec((_R, _K), lambda i: (i, 0)),
            pl.BlockSpec((_R, _K), lambda i: (i, 0)),
        ],
        out_shape=(
            jax.ShapeDtypeStruct((_NP, _K), jnp.int32),
            jax.ShapeDtypeStruct((_NP, _K), jnp.float32),
        ),
        scratch_shapes=[pltpu.VMEM((_NCK, _R, _CW), jnp.float32)],
    )(sq, sq3, s, sT3)


def _agg_sc(h, idx_flat, w):
    mesh = plsc.VectorSubcoreMesh(core_axis_name="c", subcore_axis_name="s")
    npt = _NP // 32
    nchunks = npt // _CH

    @functools.partial(
        pl.kernel,
        mesh=mesh,
        out_type=jax.ShapeDtypeStruct((_NP, 2 * _HID), jnp.float32),
        scratch_types=[
            pltpu.VMEM((_CH * _K,), jnp.int32),
            pltpu.VMEM((_CH, _K), jnp.float32),
            pltpu.VMEM((_CH * _K, _HW), jnp.float32),
            pltpu.VMEM((_CH, 2 * _HID), jnp.float32),
            pltpu.SemaphoreType.DMA,
        ],
    )
    def body(h_hbm, idx_hbm, w_hbm, out_hbm, idx_v, w_v, rows_v, out_v, sem):
        wid = lax.axis_index("s") * 2 + lax.axis_index("c")
        base = wid * npt

        def chunk(ci, carry):
            r0 = base + ci * _CH
            pltpu.sync_copy(idx_hbm.at[pl.ds(r0 * _K, _CH * _K)], idx_v)
            pltpu.async_copy(h_hbm.at[idx_v], rows_v, sem).wait()
            pltpu.sync_copy(w_hbm.at[pl.ds(r0, _CH)], w_v)
            for n in range(_CH):
                acc0 = jnp.zeros((16,), jnp.float32)
                acc1 = jnp.zeros((16,), jnp.float32)
                mx0 = jnp.full((16,), -jnp.inf, jnp.float32)
                mx1 = jnp.full((16,), -jnp.inf, jnp.float32)
                wrow = w_v[n, 0:16]
                for j in range(_K):
                    r = n * _K + j
                    wv = wrow[j]
                    a0 = rows_v[r, 0:16] * wv
                    a1 = rows_v[r, 16:32] * wv
                    acc0 = acc0 + a0
                    acc1 = acc1 + a1
                    mx0 = jnp.maximum(mx0, a0)
                    mx1 = jnp.maximum(mx1, a1)
                out_v[n, 0:16] = acc0 * (1.0 / _K)
                out_v[n, 16:32] = acc1 * (1.0 / _K)
                out_v[n, 32:48] = mx0
                out_v[n, 48:64] = mx1
            pltpu.sync_copy(out_v, out_hbm.at[pl.ds(r0, _CH)])
            return carry

        lax.fori_loop(0, nchunks, chunk, 0)

    return body(h, idx_flat, w)


def _mlp_body(x_ref, agg_ref, *refs):
    wr = [r[...] for r in refs[:-2]]
    ids_ref, p4_ref = refs[-2], refs[-1]
    (Wox, Wom, WoM, bo,
     W2x, W2h, b20, W21, b21, W22, b22, W23, b23, W24, b24,
     W3x, W3h, W3i, b30, W31, b31, W32, b32, W33, b33, W34, b34) = wr
    x = x_ref[...]
    mean = agg_ref[:, 0:_HID]
    mx = agg_ref[:, _HID:2 * _HID]

    def leaky(v):
        return jnp.where(v >= 0, v, 0.01 * v)

    x1 = leaky(_dot(x, Wox) + _dot(mean, Wom) + _dot(mx, WoM) + bo)
    a = leaky(_dot(x, W2x) + _dot(x1, W2h) + b20)
    a = leaky(_dot(a, W21) + b21)
    a = leaky(_dot(a, W22) + b22)
    a = leaky(_dot(a, W23) + b23)
    ids = _dot(a, W24) + b24
    c = leaky(_dot(x, W3x) + _dot(x1, W3h) + _dot(ids, W3i) + b30)
    c = leaky(_dot(c, W31) + b31)
    c = leaky(_dot(c, W32) + b32)
    c = leaky(_dot(c, W33) + b33)
    o3 = _dot(c, W34) + b34
    ids_ref[...] = ids
    p4_ref[...] = x[:, 11:15] + o3


def _mlp(xp, agg, weights):
    grid = (_NP // _RT,)
    full = [pl.BlockSpec(w.shape, lambda i: (0,) * w.ndim) for w in weights]
    return pl.pallas_call(
        _mlp_body,
        grid=grid,
        in_specs=[
            pl.BlockSpec((_RT, 15), lambda i: (i, 0)),
            pl.BlockSpec((_RT, 2 * _HID), lambda i: (i, 0)),
        ] + full,
        out_specs=[
            pl.BlockSpec((_RT, 6), lambda i: (i, 0)),
            pl.BlockSpec((_RT, 4), lambda i: (i, 0)),
        ],
        out_shape=(
            jax.ShapeDtypeStruct((_NP, 6), jnp.float32),
            jax.ShapeDtypeStruct((_NP, 4), jnp.float32),
        ),
    )(xp, agg, *weights)


def kernel(x, Ws, bs, Wh, bh, Wout, bout, nn2_params, nn3_params):
    xp = jnp.pad(x, ((0, _NP - _N), (0, 0)))
    Whp = jnp.pad(Wh, ((0, 0), (0, _HW - _HID)))
    bhp = jnp.pad(bh, (0, _HW - _HID))
    s, h, sq = _proj(xp, Ws, bs.reshape(1, -1), Whp, bhp.reshape(1, -1))
    sT3 = s.T.reshape(4, _NCK, _CW).swapaxes(0, 1)
    sq3 = sq.reshape(1, _NCK, _CW).swapaxes(0, 1)
    idx, w = _knn(sq, sq3, s, sT3)
    agg = _agg_sc(h, idx.reshape(-1), w)

    W20, b20 = nn2_params[0], nn2_params[1]
    W30, b30 = nn3_params[0], nn3_params[1]
    weights = [
        Wout[:15], Wout[15:15 + _HID], Wout[15 + _HID:15 + 2 * _HID],
        bout.reshape(1, -1),
        W20[:15], W20[15:], b20.reshape(1, -1),
        nn2_params[2], nn2_params[3].reshape(1, -1),
        nn2_params[4], nn2_params[5].reshape(1, -1),
        nn2_params[6], nn2_params[7].reshape(1, -1),
        nn2_params[8], nn2_params[9].reshape(1, -1),
        W30[:15], W30[15:271], W30[271:277], b30.reshape(1, -1),
        nn3_params[2], nn3_params[3].reshape(1, -1),
        nn3_params[4], nn3_params[5].reshape(1, -1),
        nn3_params[6], nn3_params[7].reshape(1, -1),
        nn3_params[8], nn3_params[9].reshape(1, -1),
    ]
    ids, p4 = _mlp(xp, agg, weights)
    return ids[:_N], p4[:_N]


# speedup vs baseline: 5.3151x; 1.0576x over previous
import functools

import jax
import jax.numpy as jnp
from jax import lax
from jax.experimental import pallas as pl
from jax.experimental.pallas import tpu as pltpu
from jax.experimental.pallas import tpu_sc as plsc

_N = 10000
_NP = 10240
_K = 16
_HID = 32
_R = 256
_CW = 1024
_NCK = _NP // _CW
_RT = 512
_CH = 8
_HW = 128


def _dot(a, b):
    return jnp.dot(a, b, preferred_element_type=jnp.float32)


def _proj_body(x_ref, ws_ref, bs_ref, wh_ref, bh_ref, s_ref, h_ref, sq_ref):
    x = x_ref[...]
    s = _dot(x, ws_ref[...]) + bs_ref[...]
    s_ref[...] = s
    h_ref[...] = _dot(x, wh_ref[...]) + bh_ref[...]
    sq_ref[...] = jnp.sum(s * s, axis=1, keepdims=True)


def _proj(xp, Ws, bs, Wh, bh):
    grid = (_NP // _RT,)
    return pl.pallas_call(
        _proj_body,
        grid=grid,
        in_specs=[
            pl.BlockSpec((_RT, 15), lambda i: (i, 0)),
            pl.BlockSpec(Ws.shape, lambda i: (0, 0)),
            pl.BlockSpec(bs.shape, lambda i: (0, 0)),
            pl.BlockSpec(Wh.shape, lambda i: (0, 0)),
            pl.BlockSpec(bh.shape, lambda i: (0, 0)),
        ],
        out_specs=[
            pl.BlockSpec((_RT, 4), lambda i: (i, 0)),
            pl.BlockSpec((_RT, _HW), lambda i: (i, 0)),
            pl.BlockSpec((_RT, 1), lambda i: (i, 0)),
        ],
        out_shape=(
            jax.ShapeDtypeStruct((_NP, 4), jnp.float32),
            jax.ShapeDtypeStruct((_NP, _HW), jnp.float32),
            jax.ShapeDtypeStruct((_NP, 1), jnp.float32),
        ),
    )(xp, Ws, bs, Wh, bh)


def _knn_body(sq_ref, sq3_ref, s_ref, st3_ref, idx_ref, w_ref, d2_ref):
    s_tile = s_ref[...]
    sq_tile = sq_ref[...]
    big = jnp.float32(jnp.inf)

    mfull = jnp.full((_R, _CW), big)
    afull = jnp.full((_R, _CW), _NP, jnp.int32)

    def casc(c, carry):
        m1, m2, m3, m4, a1, a2, a3, a4 = carry
        stc = st3_ref[c]
        colc = c * _CW + lax.broadcasted_iota(jnp.int32, (_R, _CW), 1)
        x = sq_tile + sq3_ref[c] - 2.0 * _dot(s_tile, stc)
        x = jnp.where(colc < _N, x, big)
        d2_ref[c] = x
        c1 = x < m1
        c2 = x < m2
        c3 = x < m3
        c4 = x < m4
        m4n = jnp.where(c4, jnp.where(c3, m3, x), m4)
        a4n = jnp.where(c4, jnp.where(c3, a3, colc), a4)
        m3n = jnp.where(c3, jnp.where(c2, m2, x), m3)
        a3n = jnp.where(c3, jnp.where(c2, a2, colc), a3)
        m2n = jnp.where(c2, jnp.where(c1, m1, x), m2)
        a2n = jnp.where(c2, jnp.where(c1, a1, colc), a2)
        m1n = jnp.where(c1, x, m1)
        a1n = jnp.where(c1, colc, a1)
        return (m1n, m2n, m3n, m4n, a1n, a2n, a3n, a4n)

    m1, m2, m3, m4, a1, a2, a3, a4 = lax.fori_loop(
        0, _NCK, casc,
        (mfull, mfull, mfull, mfull, afull, afull, afull, afull))

    cnt = jnp.zeros((_R, _CW), jnp.int32)
    idxs, vals = [], []
    for _ in range(_K):
        m = jnp.min(m1, axis=1, keepdims=True)
        idxt = jnp.min(jnp.where(m1 <= m, a1, _NP), axis=1, keepdims=True)
        gmask = a1 == idxt
        vals.append(m)
        idxs.append(idxt)
        m1 = jnp.where(gmask, m2, m1)
        a1 = jnp.where(gmask, a2, a1)
        m2 = jnp.where(gmask, m3, m2)
        a2 = jnp.where(gmask, a3, a2)
        m3 = jnp.where(gmask, m4, m3)
        a3 = jnp.where(gmask, a4, a3)
        m4 = jnp.where(gmask, big, m4)
        a4 = jnp.where(gmask, _NP, a4)
        cnt = cnt + gmask.astype(jnp.int32)
    idx_ref[...] = jnp.concatenate(idxs, axis=1)
    w_ref[...] = jnp.exp(-10.0 * jnp.maximum(jnp.concatenate(vals, axis=1),
                                             0.0))

    @pl.when(jnp.any(cnt >= 4))
    def _slow():
        sidxs, svals = [], []
        sel_prev = jnp.full((_R, 1), -1, jnp.int32)
        for t in range(_K):
            def p_a(c, mm, sel_prev=sel_prev):
                d2c = d2_ref[c]
                colc = c * _CW + lax.broadcasted_iota(jnp.int32,
                                                      (_R, _CW), 1)
                d2c = jnp.where(colc == sel_prev, big, d2c)
                d2_ref[c] = d2c
                return jnp.minimum(mm, jnp.min(d2c, axis=1, keepdims=True))

            mm = lax.fori_loop(0, _NCK, p_a, jnp.full((_R, 1), big))

            def p_b(c, sel, mm=mm):
                d2c = d2_ref[c]
                colc = c * _CW + lax.broadcasted_iota(jnp.int32,
                                                      (_R, _CW), 1)
                selc = jnp.min(jnp.where(d2c <= mm, colc, _NP), axis=1,
                               keepdims=True)
                return jnp.minimum(sel, selc)

            sel = lax.fori_loop(0, _NCK, p_b,
                                jnp.full((_R, 1), _NP, jnp.int32))
            svals.append(mm)
            sidxs.append(sel)
            sel_prev = sel
        idx_ref[...] = jnp.concatenate(sidxs, axis=1)
        w_ref[...] = jnp.exp(
            -10.0 * jnp.maximum(jnp.concatenate(svals, axis=1), 0.0))


def _knn(sq, sq3, s, sT3):
    grid = (_NP // _R,)
    return pl.pallas_call(
        _knn_body,
        grid=grid,
        in_specs=[
            pl.BlockSpec((_R, 1), lambda i: (i, 0)),
            pl.BlockSpec((_NCK, 1, _CW), lambda i: (0, 0, 0)),
            pl.BlockSpec((_R, 4), lambda i: (i, 0)),
            pl.BlockSpec((_NCK, 4, _CW), lambda i: (0, 0, 0)),
        ],
        out_specs=[
            pl.BlockSpec((_R, _K), lambda i: (i, 0)),
            pl.BlockSpec((_R, _K), lambda i: (i, 0)),
        ],
        out_shape=(
            jax.ShapeDtypeStruct((_NP, _K), jnp.int32),
            jax.ShapeDtypeStruct((_NP, _K), jnp.float32),
        ),
        scratch_shapes=[pltpu.VMEM((_NCK, _R, _CW), jnp.float32)],
    )(sq, sq3, s, sT3)


def _agg_sc(h, idx_flat, w):
    mesh = plsc.VectorSubcoreMesh(core_axis_name="c", subcore_axis_name="s")
    npt = _NP // 32
    nchunks = npt // _CH

    @functools.partial(
        pl.kernel,
        mesh=mesh,
        out_type=jax.ShapeDtypeStruct((_NP, 2 * _HID), jnp.float32),
        scratch_types=[
            pltpu.VMEM((_CH * _K,), jnp.int32),
            pltpu.VMEM((_CH, _K), jnp.float32),
            pltpu.VMEM((_CH * _K, _HW), jnp.float32),
            pltpu.VMEM((_CH, 2 * _HID), jnp.float32),
            pltpu.SemaphoreType.DMA,
        ],
    )
    def body(h_hbm, idx_hbm, w_hbm, out_hbm, idx_v, w_v, rows_v, out_v, sem):
        wid = lax.axis_index("s") * 2 + lax.axis_index("c")
        base = wid * npt

        def chunk(ci, carry):
            r0 = base + ci * _CH
            pltpu.sync_copy(idx_hbm.at[pl.ds(r0 * _K, _CH * _K)], idx_v)
            pltpu.async_copy(h_hbm.at[idx_v], rows_v, sem).wait()
            pltpu.sync_copy(w_hbm.at[pl.ds(r0, _CH)], w_v)
            for n in range(_CH):
                acc0 = jnp.zeros((16,), jnp.float32)
                acc1 = jnp.zeros((16,), jnp.float32)
                mx0 = jnp.full((16,), -jnp.inf, jnp.float32)
                mx1 = jnp.full((16,), -jnp.inf, jnp.float32)
                wrow = w_v[n, 0:16]
                for j in range(_K):
                    r = n * _K + j
                    wv = wrow[j]
                    a0 = rows_v[r, 0:16] * wv
                    a1 = rows_v[r, 16:32] * wv
                    acc0 = acc0 + a0
                    acc1 = acc1 + a1
                    mx0 = jnp.maximum(mx0, a0)
                    mx1 = jnp.maximum(mx1, a1)
                out_v[n, 0:16] = acc0 * (1.0 / _K)
                out_v[n, 16:32] = acc1 * (1.0 / _K)
                out_v[n, 32:48] = mx0
                out_v[n, 48:64] = mx1
            pltpu.sync_copy(out_v, out_hbm.at[pl.ds(r0, _CH)])
            return carry

        lax.fori_loop(0, nchunks, chunk, 0)

    return body(h, idx_flat, w)


def _mlp_body(x_ref, agg_ref, *refs):
    wr = [r[...] for r in refs[:-2]]
    ids_ref, p4_ref = refs[-2], refs[-1]
    (Wox, Wom, WoM, bo,
     W2x, W2h, b20, W21, b21, W22, b22, W23, b23, W24, b24,
     W3x, W3h, W3i, b30, W31, b31, W32, b32, W33, b33, W34, b34) = wr
    x = x_ref[...]
    mean = agg_ref[:, 0:_HID]
    mx = agg_ref[:, _HID:2 * _HID]

    def leaky(v):
        return jnp.where(v >= 0, v, 0.01 * v)

    x1 = leaky(_dot(x, Wox) + _dot(mean, Wom) + _dot(mx, WoM) + bo)
    a = leaky(_dot(x, W2x) + _dot(x1, W2h) + b20)
    a = leaky(_dot(a, W21) + b21)
    a = leaky(_dot(a, W22) + b22)
    a = leaky(_dot(a, W23) + b23)
    ids = _dot(a, W24) + b24
    c = leaky(_dot(x, W3x) + _dot(x1, W3h) + _dot(ids, W3i) + b30)
    c = leaky(_dot(c, W31) + b31)
    c = leaky(_dot(c, W32) + b32)
    c = leaky(_dot(c, W33) + b33)
    o3 = _dot(c, W34) + b34
    ids_ref[...] = ids
    p4_ref[...] = x[:, 11:15] + o3


def _mlp(xp, agg, weights):
    grid = (_NP // _RT,)
    full = [pl.BlockSpec(w.shape, lambda i: (0,) * w.ndim) for w in weights]
    return pl.pallas_call(
        _mlp_body,
        grid=grid,
        in_specs=[
            pl.BlockSpec((_RT, 15), lambda i: (i, 0)),
            pl.BlockSpec((_RT, 2 * _HID), lambda i: (i, 0)),
        ] + full,
        out_specs=[
            pl.BlockSpec((_RT, 6), lambda i: (i, 0)),
            pl.BlockSpec((_RT, 4), lambda i: (i, 0)),
        ],
        out_shape=(
            jax.ShapeDtypeStruct((_NP, 6), jnp.float32),
            jax.ShapeDtypeStruct((_NP, 4), jnp.float32),
        ),
    )(xp, agg, *weights)


def kernel(x, Ws, bs, Wh, bh, Wout, bout, nn2_params, nn3_params):
    xp = jnp.pad(x, ((0, _NP - _N), (0, 0)))
    Whp = jnp.pad(Wh, ((0, 0), (0, _HW - _HID)))
    bhp = jnp.pad(bh, (0, _HW - _HID))
    s, h, sq = _proj(xp, Ws, bs.reshape(1, -1), Whp, bhp.reshape(1, -1))
    sT3 = s.T.reshape(4, _NCK, _CW).swapaxes(0, 1)
    sq3 = sq.reshape(1, _NCK, _CW).swapaxes(0, 1)
    idx, w = _knn(sq, sq3, s, sT3)
    agg = _agg_sc(h, idx.reshape(-1), w)

    W20, b20 = nn2_params[0], nn2_params[1]
    W30, b30 = nn3_params[0], nn3_params[1]
    weights = [
        Wout[:15], Wout[15:15 + _HID], Wout[15 + _HID:15 + 2 * _HID],
        bout.reshape(1, -1),
        W20[:15], W20[15:], b20.reshape(1, -1),
        nn2_params[2], nn2_params[3].reshape(1, -1),
        nn2_params[4], nn2_params[5].reshape(1, -1),
        nn2_params[6], nn2_params[7].reshape(1, -1),
        nn2_params[8], nn2_params[9].reshape(1, -1),
        W30[:15], W30[15:271], W30[271:277], b30.reshape(1, -1),
        nn3_params[2], nn3_params[3].reshape(1, -1),
        nn3_params[4], nn3_params[5].reshape(1, -1),
        nn3_params[6], nn3_params[7].reshape(1, -1),
        nn3_params[8], nn3_params[9].reshape(1, -1),
    ]
    ids, p4 = _mlp(xp, agg, weights)
    return ids[:_N], p4[:_N]
